# Initial kernel scaffold; baseline (speedup 1.0000x reference)
import jax
import jax.numpy as jnp
from jax.experimental import pallas as pl


def kernel(X, weights):
    raise NotImplementedError("write your pallas kernel here")


# speedup vs baseline: 1.9462x; 1.9462x over previous
import functools

import jax
import jax.numpy as jnp
from jax import lax
from jax.experimental import pallas as pl
from jax.experimental.pallas import tpu as pltpu
from jax.experimental.pallas import tpu_sc as plsc

PADDING_IDX = 1
B = 4
S = 4096
D = 1024

NC = 2
NS = 16
NW = NC * NS

ROWS = B * S
RPW = ROWS // NW
CHUNK = 32
NCHUNK = RPW // CHUNK
LANES = 16

_mesh = plsc.VectorSubcoreMesh(core_axis_name="c", subcore_axis_name="s")


@functools.partial(
    pl.kernel,
    out_type=jax.ShapeDtypeStruct((ROWS, D), jnp.float32),
    mesh=_mesh,
    scratch_types=[
        pltpu.VMEM((RPW,), jnp.int32),
        pltpu.VMEM((RPW,), jnp.int32),
        pltpu.VMEM((2, CHUNK, D), jnp.float32),
        pltpu.SemaphoreType.DMA,
        pltpu.SemaphoreType.DMA,
    ],
)
def _sc_embed(x_hbm, w_hbm, out_hbm, x_v, idx_v, rows_v, sem0, sem1):
    wid = lax.axis_index("c") * NS + lax.axis_index("s")
    base = wid * RPW
    s0 = (wid % (S // RPW)) * RPW

    pltpu.sync_copy(x_hbm.at[pl.ds(base, RPW)], x_v)

    iota = lax.broadcasted_iota(jnp.int32, (LANES,), 0)
    for i in range(RPW // LANES):
        tok = x_v[pl.ds(i * LANES, LANES)]
        pos = iota + (s0 + i * LANES + PADDING_IDX + 1)
        idx_v[pl.ds(i * LANES, LANES)] = jnp.where(
            tok == PADDING_IDX, PADDING_IDX, pos)

    sems = (sem0, sem1)

    def gather(c, slot):
        return pltpu.make_async_copy(
            w_hbm.at[idx_v.at[pl.ds(c * CHUNK, CHUNK)]],
            rows_v.at[slot], sems[slot])

    gather(0, 0).start()

    @pl.loop(0, NCHUNK, step=2)
    def _chunks(g):
        for b in range(2):
            cur = g + b
            nxt = cur + 1

            @pl.when(nxt < NCHUNK)
            def _():
                gather(nxt, b ^ 1).start()

            gather(cur, b).wait()
            pltpu.sync_copy(rows_v.at[b],
                            out_hbm.at[pl.ds(base + cur * CHUNK, CHUNK)])


def kernel(X, weights):
    out = _sc_embed(X.reshape(ROWS), weights)
    return out.reshape(B, S, D)


# speedup vs baseline: 1.9851x; 1.0200x over previous
import functools

import jax
import jax.numpy as jnp
from jax import lax
from jax.experimental import pallas as pl
from jax.experimental.pallas import tpu as pltpu
from jax.experimental.pallas import tpu_sc as plsc

PADDING_IDX = 1
B = 4
S = 4096
D = 1024

NC = 2
NS = 16
NW = NC * NS

ROWS = B * S
RPW = ROWS // NW
CHUNK = 32
NCHUNK = RPW // CHUNK
NBUF = 3
LAG = 2
LANES = 16

_mesh = plsc.VectorSubcoreMesh(core_axis_name="c", subcore_axis_name="s")


@functools.partial(
    pl.kernel,
    out_type=jax.ShapeDtypeStruct((ROWS, D), jnp.float32),
    mesh=_mesh,
    scratch_types=[
        pltpu.VMEM((RPW,), jnp.int32),
        pltpu.VMEM((RPW,), jnp.int32),
        pltpu.VMEM((NBUF, CHUNK, D), jnp.float32),
        pltpu.SemaphoreType.DMA,
        pltpu.SemaphoreType.DMA,
        pltpu.SemaphoreType.DMA,
        pltpu.SemaphoreType.DMA,
        pltpu.SemaphoreType.DMA,
        pltpu.SemaphoreType.DMA,
    ],
)
def _sc_embed(x_hbm, w_hbm, out_hbm, x_v, idx_v, rows_v,
              g0, g1, g2, s0_, s1_, s2_):
    gsem = (g0, g1, g2)
    ssem = (s0_, s1_, s2_)

    wid = lax.axis_index("c") * NS + lax.axis_index("s")
    base = wid * RPW
    seq0 = (wid % (S // RPW)) * RPW

    pltpu.sync_copy(x_hbm.at[pl.ds(base, RPW)], x_v)

    iota = lax.broadcasted_iota(jnp.int32, (LANES,), 0)
    for i in range(RPW // LANES):
        tok = x_v[pl.ds(i * LANES, LANES)]
        pos = iota + (seq0 + i * LANES + PADDING_IDX + 1)
        idx_v[pl.ds(i * LANES, LANES)] = jnp.where(
            tok == PADDING_IDX, PADDING_IDX, pos)

    def start_gather(c, sl):
        pltpu.async_copy(w_hbm.at[idx_v.at[pl.ds(c * CHUNK, CHUNK)]],
                         rows_v.at[sl], gsem[sl])

    def wait_gather(sl):
        pltpu.make_async_copy(w_hbm.at[pl.ds(0, CHUNK)],
                              rows_v.at[sl], gsem[sl]).wait()

    def start_scatter(c, sl):
        pltpu.async_copy(rows_v.at[sl],
                         out_hbm.at[pl.ds(base + c * CHUNK, CHUNK)], ssem[sl])

    def wait_scatter(sl):
        pltpu.make_async_copy(rows_v.at[sl],
                              out_hbm.at[pl.ds(0, CHUNK)], ssem[sl]).wait()

    @pl.loop(0, NCHUNK + LAG, step=NBUF)
    def _pipe(g):
        for j in range(NBUF):
            c = g + j

            @pl.when(c < NCHUNK)
            def _():
                @pl.when(c >= NBUF)
                def _():
                    wait_scatter(j)
                start_gather(c, j)

            d = c - LAG
            sl_d = (j - LAG) % NBUF

            @pl.when(jnp.logical_and(d >= 0, d < NCHUNK))
            def _():
                wait_gather(sl_d)
                start_scatter(d, sl_d)

    for j in range(NBUF):
        wait_scatter(j)


def kernel(X, weights):
    out = _sc_embed(X.reshape(ROWS), weights)
    return out.reshape(B, S, D)


# speedup vs baseline: 2.6968x; 1.3585x over previous
import functools

import jax
import jax.numpy as jnp
from jax import lax
from jax.experimental import pallas as pl
from jax.experimental.pallas import tpu as pltpu
from jax.experimental.pallas import tpu_sc as plsc

PADDING_IDX = 1
B = 4
S = 4096
D = 1024

NC = 2
NS = 16
NW = NC * NS

SPW = S // NW
CHUNK = 32
NCHUNK = SPW // CHUNK
LANES = 16
NGRP = SPW // LANES

_mesh = plsc.VectorSubcoreMesh(core_axis_name="c", subcore_axis_name="s")


@functools.partial(
    pl.kernel,
    out_type=jax.ShapeDtypeStruct((B * S, D), jnp.float32),
    mesh=_mesh,
    scratch_types=[
        pltpu.VMEM((B * SPW,), jnp.int32),
        pltpu.VMEM((B * SPW,), jnp.int32),
        pltpu.VMEM((SPW,), jnp.int32),
        pltpu.VMEM((3, CHUNK, D), jnp.float32),
        pltpu.VMEM((B * NCHUNK,), jnp.int32),
        pltpu.SemaphoreType.DMA,
        pltpu.SemaphoreType.DMA,
        pltpu.SemaphoreType.DMA,
        pltpu.SemaphoreType.DMA,
        pltpu.SemaphoreType.DMA,
        pltpu.SemaphoreType.DMA,
        pltpu.SemaphoreType.DMA,
    ],
)
def _sc_embed(x_hbm, w_hbm, flag_hbm, out_hbm, x_v, idx_v, cidx_v, clean_v,
              flag_v, xsem, g0, g1, g2, s0_, s1_, s2_):
    gsem = (g0, g1, g2)
    ssem = (s0_, s1_, s2_)
    fix_v = clean_v.at[0]

    wid = lax.axis_index("c") * NS + lax.axis_index("s")
    s0 = wid * SPW

    def start_gather(c, sl):
        pltpu.async_copy(w_hbm.at[cidx_v.at[pl.ds(c * CHUNK, CHUNK)]],
                         clean_v.at[sl], gsem[sl])

    def wait_gather(sl):
        pltpu.make_async_copy(w_hbm.at[pl.ds(0, CHUNK)],
                              clean_v.at[sl], gsem[sl]).wait()

    def start_scatters(c, sl):
        for b in range(B):
            pltpu.async_copy(
                clean_v.at[sl],
                out_hbm.at[pl.ds(b * S + s0 + c * CHUNK, CHUNK)], ssem[sl])

    def drain_scatters(sl):
        for _ in range(B):
            pltpu.make_async_copy(clean_v.at[sl],
                                  out_hbm.at[pl.ds(0, CHUNK)],
                                  ssem[sl]).wait()

    iota = lax.broadcasted_iota(jnp.int32, (LANES,), 0)
    for g in range(NGRP):
        cidx_v[pl.ds(g * LANES, LANES)] = iota + (s0 + g * LANES
                                                  + PADDING_IDX + 1)
    start_gather(0, 0)
    start_gather(1, 1)
    start_gather(2, 2)

    pltpu.async_copy(flag_hbm.at[wid], flag_v, xsem)
    for b in range(B):
        pltpu.async_copy(x_hbm.at[pl.ds(b * S + s0, SPW)],
                         x_v.at[pl.ds(b * SPW, SPW)], xsem)
    pltpu.make_async_copy(flag_hbm.at[wid], flag_v, xsem).wait()
    for b in range(B):
        pltpu.make_async_copy(x_hbm.at[pl.ds(0, SPW)],
                              x_v.at[pl.ds(0, SPW)], xsem).wait()

    for b in range(B):
        for g in range(NGRP):
            tok = x_v[pl.ds(b * SPW + g * LANES, LANES)]
            pos = iota + (s0 + g * LANES + PADDING_IDX + 1)
            idx_v[pl.ds(b * SPW + g * LANES, LANES)] = jnp.where(
                tok == PADDING_IDX, PADDING_IDX, pos)

    for c in range(NCHUNK):
        sl = c % 3
        if c >= 2 and c + 1 < NCHUNK:
            drain_scatters((c + 1) % 3)
            start_gather(c + 1, (c + 1) % 3)
        wait_gather(sl)
        start_scatters(c, sl)
    for c in range(max(0, NCHUNK - 3), NCHUNK):
        drain_scatters(c % 3)

    flags = flag_v[pl.ds(0, B * NCHUNK)]
    for b in range(B):
        for c in range(NCHUNK):
            @pl.when(flags[b * NCHUNK + c] > 0)
            def _(b=b, c=c):
                pltpu.async_copy(
                    w_hbm.at[idx_v.at[pl.ds(b * SPW + c * CHUNK, CHUNK)]],
                    fix_v, xsem)
                pltpu.make_async_copy(w_hbm.at[pl.ds(0, CHUNK)],
                                      fix_v, xsem).wait()
                pltpu.sync_copy(
                    fix_v,
                    out_hbm.at[pl.ds(b * S + s0 + c * CHUNK, CHUNK)])


def kernel(X, weights):
    dirty = jnp.any(
        X.reshape(B, NW, NCHUNK, CHUNK) == PADDING_IDX, axis=-1)
    flags = dirty.transpose(1, 0, 2).reshape(NW, B * NCHUNK).astype(jnp.int32)
    out = _sc_embed(X.reshape(B * S), weights, flags)
    return out.reshape(B, S, D)
